# baseline (speedup 1.0000x reference)
import jax
import jax.numpy as jnp
from jax import lax
from jax.experimental import pallas as pl
from jax.experimental.pallas import tpu as pltpu

B, N, NFEAT, NHID, NCLASS = 8, 2048, 128, 16, 128


def _gcn_body(x_ref, adj_hbm, w1_ref, b1_ref, w2_ref, b2_ref, wfc_ref,
              bfc_ref, out_ref, abuf, abf, s2buf, sem):
    b = pl.program_id(0)

    @pl.when(b == 0)
    def _():
        pltpu.make_async_copy(adj_hbm.at[0], abuf.at[0], sem.at[0]).start()
        pltpu.make_async_copy(adj_hbm.at[1], abuf.at[1], sem.at[1]).start()

    @pl.when(jnp.logical_and(b >= 1, b + 1 < B))
    def _():
        s = (b + 1) % 2
        pltpu.make_async_copy(adj_hbm.at[b + 1], abuf.at[s], sem.at[s]).start()

    @pl.when(b < B)
    def _():
        s = b % 2
        pltpu.make_async_copy(adj_hbm.at[b], abuf.at[s], sem.at[s]).wait()
        abf[s] = abuf[s].astype(jnp.bfloat16)
        a = abf[s]
        s1 = jnp.dot(x_ref[0], w1_ref[...],
                     preferred_element_type=jnp.float32)
        hT = jnp.maximum(
            lax.dot_general(s1.astype(jnp.bfloat16), a,
                            (((0,), (1,)), ((), ())),
                            preferred_element_type=jnp.float32)
            + b1_ref[...], 0.0)
        s2buf[s] = lax.dot_general(
            w2_ref[...], hT, (((0,), (0,)), ((), ())),
            preferred_element_type=jnp.float32)

    @pl.when(b >= 1)
    def _():
        s = (b - 1) % 2
        a = abf[s]
        s2 = s2buf[s]
        g_row = lax.dot_general(s2.astype(jnp.bfloat16), a,
                                (((1,), (1,)), ((), ())),
                                preferred_element_type=jnp.float32) \
            + b2_ref[...]
        logits = lax.dot_general(g_row, wfc_ref[...],
                                 (((1,), (1,)), ((), ())),
                                 preferred_element_type=jnp.float32) \
            + bfc_ref[...]
        m = jnp.max(logits, axis=1, keepdims=True)
        shifted = logits - m
        lse = jnp.log(jnp.sum(jnp.exp(shifted), axis=1, keepdims=True))
        out_ref[0] = shifted - lse


def kernel(x, adj, W1, b1, W2, b2, Wfc, bfc):
    out = pl.pallas_call(
        _gcn_body,
        grid=(B + 1,),
        in_specs=[
            pl.BlockSpec((1, N, NFEAT),
                         lambda b: (jnp.minimum(b, B - 1), 0, 0)),
            pl.BlockSpec(memory_space=pltpu.MemorySpace.HBM),
            pl.BlockSpec((NFEAT, NHID), lambda b: (0, 0)),
            pl.BlockSpec((NHID, 1), lambda b: (0, 0)),
            pl.BlockSpec((NHID, 1), lambda b: (0, 0)),
            pl.BlockSpec((1, 1), lambda b: (0, 0)),
            pl.BlockSpec((NCLASS, N), lambda b: (0, 0)),
            pl.BlockSpec((1, NCLASS), lambda b: (0, 0)),
        ],
        out_specs=pl.BlockSpec((1, 1, NCLASS),
                               lambda b: (jnp.maximum(b - 1, 0), 0, 0)),
        out_shape=jax.ShapeDtypeStruct((B, 1, NCLASS), jnp.float32),
        scratch_shapes=[
            pltpu.VMEM((2, N, N), jnp.float32),
            pltpu.VMEM((2, N, N), jnp.bfloat16),
            pltpu.VMEM((2, 1, N), jnp.float32),
            pltpu.SemaphoreType.DMA((2,)),
        ],
        compiler_params=pltpu.CompilerParams(
            dimension_semantics=("arbitrary",)),
    )(x, adj, W1, b1.reshape(NHID, 1), W2, b2.reshape(1, 1), Wfc,
      bfc.reshape(1, NCLASS))
    return out[:, 0, :]


# speedup vs baseline: 1.0263x; 1.0263x over previous
import jax
import jax.numpy as jnp
from jax import lax
from jax.experimental import pallas as pl
from jax.experimental.pallas import tpu as pltpu

B, N, NFEAT, NHID, NCLASS = 8, 2048, 128, 16, 128


def _gcn_body(x_ref, adj_hbm, w1_ref, b1_ref, w2_ref, b2_ref, wfc_ref,
              bfc_ref, out_ref, abuf, abf, s2buf, sem):
    b = pl.program_id(0)

    def pass1(bb):
        s = bb % 2
        pltpu.make_async_copy(adj_hbm.at[bb], abuf.at[s], sem.at[s]).wait()
        abf[s] = abuf[s].astype(jnp.bfloat16)
        s1 = jnp.dot(x_ref[0], w1_ref[...],
                     preferred_element_type=jnp.float32)
        hT = jnp.maximum(
            lax.dot_general(s1.astype(jnp.bfloat16), abf[s],
                            (((0,), (1,)), ((), ())),
                            preferred_element_type=jnp.float32)
            + b1_ref[...], 0.0)
        s2buf[s] = lax.dot_general(
            w2_ref[...], hT, (((0,), (0,)), ((), ())),
            preferred_element_type=jnp.float32)

    def pass2(bb):
        s = bb % 2
        g_row = lax.dot_general(s2buf[s].astype(jnp.bfloat16), abf[s],
                                (((1,), (1,)), ((), ())),
                                preferred_element_type=jnp.float32) \
            + b2_ref[...]
        logits = lax.dot_general(g_row, wfc_ref[...],
                                 (((1,), (1,)), ((), ())),
                                 preferred_element_type=jnp.float32) \
            + bfc_ref[...]
        m = jnp.max(logits, axis=1, keepdims=True)
        shifted = logits - m
        lse = jnp.log(jnp.sum(jnp.exp(shifted), axis=1, keepdims=True))
        out_ref[0] = shifted - lse

    @pl.when(b == 0)
    def _():
        pltpu.make_async_copy(adj_hbm.at[0], abuf.at[0], sem.at[0]).start()
        pltpu.make_async_copy(adj_hbm.at[1], abuf.at[1], sem.at[1]).start()
        pass1(0)

    @pl.when(jnp.logical_and(b >= 1, b + 1 < B))
    def _():
        s = (b + 1) % 2
        pltpu.make_async_copy(adj_hbm.at[b + 1], abuf.at[s], sem.at[s]).start()

    @pl.when(jnp.logical_and(b >= 1, b < B))
    def _():
        pass1(b)
        pass2(b - 1)

    @pl.when(b == B)
    def _():
        pass2(B - 1)


def kernel(x, adj, W1, b1, W2, b2, Wfc, bfc):
    out = pl.pallas_call(
        _gcn_body,
        grid=(B + 1,),
        in_specs=[
            pl.BlockSpec((1, N, NFEAT),
                         lambda b: (jnp.minimum(b, B - 1), 0, 0)),
            pl.BlockSpec(memory_space=pltpu.MemorySpace.HBM),
            pl.BlockSpec((NFEAT, NHID), lambda b: (0, 0)),
            pl.BlockSpec((NHID, 1), lambda b: (0, 0)),
            pl.BlockSpec((NHID, 1), lambda b: (0, 0)),
            pl.BlockSpec((1, 1), lambda b: (0, 0)),
            pl.BlockSpec((NCLASS, N), lambda b: (0, 0)),
            pl.BlockSpec((1, NCLASS), lambda b: (0, 0)),
        ],
        out_specs=pl.BlockSpec((1, 1, NCLASS),
                               lambda b: (jnp.maximum(b - 1, 0), 0, 0)),
        out_shape=jax.ShapeDtypeStruct((B, 1, NCLASS), jnp.float32),
        scratch_shapes=[
            pltpu.VMEM((2, N, N), jnp.float32),
            pltpu.VMEM((2, N, N), jnp.bfloat16),
            pltpu.VMEM((2, 1, N), jnp.float32),
            pltpu.SemaphoreType.DMA((2,)),
        ],
        compiler_params=pltpu.CompilerParams(
            dimension_semantics=("arbitrary",)),
    )(x, adj, W1, b1.reshape(NHID, 1), W2, b2.reshape(1, 1), Wfc,
      bfc.reshape(1, NCLASS))
    return out[:, 0, :]


# speedup vs baseline: 1.1056x; 1.0774x over previous
import jax
import jax.numpy as jnp
from jax import lax
from jax.experimental import pallas as pl
from jax.experimental.pallas import tpu as pltpu

B, N, NFEAT, NHID, NCLASS = 8, 2048, 128, 16, 128
NS = 1
RC = N // NS


def _gcn_body(*refs):
    (x_ref, *a_refs) = refs[:1 + NS]
    (w1_ref, b1_ref, w2_ref, b2_ref, wfc_ref, bfc_ref, out_ref) = refs[1 + NS:]
    xb = x_ref[0]
    s1 = jnp.dot(xb, w1_ref[...],
                 preferred_element_type=jnp.float32)
    s1b = s1.astype(jnp.bfloat16)
    hTs = [
        jnp.maximum(
            lax.dot_general(s1b, a_ref[0].astype(jnp.bfloat16),
                            (((0,), (1,)), ((), ())),
                            preferred_element_type=jnp.float32)
            + b1_ref[...], 0.0)
        for a_ref in a_refs
    ]
    s2_row = jnp.concatenate([
        lax.dot_general(w2_ref[...], hT, (((0,), (0,)), ((), ())),
                        preferred_element_type=jnp.float32)
        for hT in hTs], axis=1)
    s2b = s2_row.astype(jnp.bfloat16)
    logits = bfc_ref[...]
    for s, a_ref in enumerate(a_refs):
        g_part = lax.dot_general(s2b, a_ref[0].astype(jnp.bfloat16),
                                 (((1,), (1,)), ((), ())),
                                 preferred_element_type=jnp.float32) \
            + b2_ref[...]
        logits = logits + lax.dot_general(
            g_part, wfc_ref[:, s * RC:(s + 1) * RC],
            (((1,), (1,)), ((), ())),
            preferred_element_type=jnp.float32)
    m = jnp.max(logits, axis=1, keepdims=True)
    shifted = logits - m
    lse = jnp.log(jnp.sum(jnp.exp(shifted), axis=1, keepdims=True))
    out_ref[0] = shifted - lse


def kernel(x, adj, W1, b1, W2, b2, Wfc, bfc):
    adj_specs = [
        pl.BlockSpec((1, RC, N), lambda b, s=s: (b, s, 0)) for s in range(NS)
    ]
    out = pl.pallas_call(
        _gcn_body,
        grid=(B,),
        in_specs=[pl.BlockSpec((1, N, NFEAT), lambda b: (b, 0, 0))]
        + adj_specs
        + [
            pl.BlockSpec((NFEAT, NHID), lambda b: (0, 0)),
            pl.BlockSpec((NHID, 1), lambda b: (0, 0)),
            pl.BlockSpec((NHID, 1), lambda b: (0, 0)),
            pl.BlockSpec((1, 1), lambda b: (0, 0)),
            pl.BlockSpec((NCLASS, N), lambda b: (0, 0)),
            pl.BlockSpec((1, NCLASS), lambda b: (0, 0)),
        ],
        out_specs=pl.BlockSpec((1, 1, NCLASS), lambda b: (b, 0, 0)),
        out_shape=jax.ShapeDtypeStruct((B, 1, NCLASS), jnp.float32),
        compiler_params=pltpu.CompilerParams(
            dimension_semantics=("arbitrary",)),
    )(x, *([adj] * NS), W1, b1.reshape(NHID, 1), W2, b2.reshape(1, 1), Wfc,
      bfc.reshape(1, NCLASS))
    return out[:, 0, :]
